# baseline (speedup 1.0000x reference)
import functools

import jax
import jax.numpy as jnp
from jax.experimental import pallas as pl
from jax.experimental.pallas import tpu as pltpu


_VMEM_LIMIT = 48 * 1024 * 1024
_EPS = 1e-5


def _round_up(x, m):
    return (x + m - 1) // m * m


def _pick_tiles(M, K, C):
    tk = min(K, 4096)
    w_bytes = 2 * (tk * C * 2)
    budget = 30 * 1024 * 1024 - w_bytes
    per_row = 2 * (tk * 2) + C * 4 + 2 * (C * 2)
    tm = budget // per_row
    tm = max(8, min(tm, 4096, _round_up(M, 8)))
    tm = (tm // 8) * 8
    while M / tm < 8 and tm > 64:
        tm = max(64, (tm // 2 // 8) * 8)
    return tm, tk


def _extract_patches(x_nhwc, kh, kw, stride):
    return jax.lax.conv_general_dilated_patches(
        x_nhwc, filter_shape=(kh, kw), window_strides=(stride, stride),
        padding="VALID", dimension_numbers=("NHWC", "HWIO", "NHWC"))


def _layer_kernel(x_ref, w_ref, scale_ref, shift_ref,
                  o_ref, s_ref, sq_ref, acc_ref, *,
                  m_true, tm, need_mask, apply_bn):
    i = pl.program_id(0)
    k = pl.program_id(1)

    @pl.when(k == 0)
    def _():
        acc_ref[...] = jnp.zeros_like(acc_ref)

    x = x_ref[...]
    if apply_bn:
        y = x.astype(jnp.float32) * scale_ref[...] + shift_ref[...]
        x = jnp.where(y >= 0.0, y, 0.2 * y).astype(jnp.bfloat16)
    acc_ref[...] += jnp.dot(x, w_ref[...], preferred_element_type=jnp.float32)

    @pl.when(k == pl.num_programs(1) - 1)
    def _():
        y = acc_ref[...]
        if need_mask:
            row = i * tm + jax.lax.broadcasted_iota(jnp.int32, y.shape, 0)
            y = jnp.where(row < m_true, y, 0.0)
        s_ref[...] = jnp.sum(y, axis=0, keepdims=True)[None]
        sq_ref[...] = jnp.sum(y * y, axis=0, keepdims=True)[None]
        o_ref[...] = y.astype(o_ref.dtype)


def _fused_layer(patches, w_mat, scale_k, shift_k):
    M, K = patches.shape
    C = w_mat.shape[1]
    tm, tk = _pick_tiles(M, K, C)
    nm = pl.cdiv(M, tm)
    nk = K // tk
    apply_bn = scale_k is not None
    if not apply_bn:
        scale_k = jnp.ones((1, K), jnp.float32)
        shift_k = jnp.zeros((1, K), jnp.float32)
    kern = functools.partial(_layer_kernel, m_true=M, tm=tm,
                             need_mask=(M % tm != 0), apply_bn=apply_bn)
    pre, s, sq = pl.pallas_call(
        kern,
        out_shape=(jax.ShapeDtypeStruct((M, C), jnp.bfloat16),
                   jax.ShapeDtypeStruct((nm, 1, C), jnp.float32),
                   jax.ShapeDtypeStruct((nm, 1, C), jnp.float32)),
        grid_spec=pltpu.PrefetchScalarGridSpec(
            num_scalar_prefetch=0,
            grid=(nm, nk),
            in_specs=[pl.BlockSpec((tm, tk), lambda i, k: (i, k)),
                      pl.BlockSpec((tk, C), lambda i, k: (k, 0)),
                      pl.BlockSpec((1, tk), lambda i, k: (0, k)),
                      pl.BlockSpec((1, tk), lambda i, k: (0, k))],
            out_specs=(pl.BlockSpec((tm, C), lambda i, k: (i, 0)),
                       pl.BlockSpec((1, 1, C), lambda i, k: (i, 0, 0)),
                       pl.BlockSpec((1, 1, C), lambda i, k: (i, 0, 0))),
            scratch_shapes=[pltpu.VMEM((tm, C), jnp.float32)],
        ),
        compiler_params=pltpu.CompilerParams(
            dimension_semantics=("parallel", "arbitrary"),
            vmem_limit_bytes=_VMEM_LIMIT),
    )(patches, w_mat, scale_k, shift_k)
    return pre, s, sq


def _bn_coeffs(s_part, sq_part, gamma, beta, M):
    s = jnp.sum(s_part, axis=(0, 1))
    sq = jnp.sum(sq_part, axis=(0, 1))
    inv_m = 1.0 / M
    mean = s * inv_m
    var = jnp.maximum(sq * inv_m - mean * mean, 0.0)
    scale = gamma * jax.lax.rsqrt(var + _EPS)
    shift = beta - mean * scale
    return scale, shift


def _fold_k(vec, C):
    return jnp.broadcast_to(vec[:, None], (C, 16)).reshape(1, 16 * C)


def _tail_kernel(x_ref, sc_ref, sh_ref, w_ref, b_ref, o_ref):
    y = x_ref[...].astype(jnp.float32) * sc_ref[...] + sh_ref[...]
    a = jnp.where(y >= 0.0, y, 0.2 * y)
    prod = a * w_ref[...]
    t = jnp.sum(prod, axis=2, keepdims=True)
    t = jnp.sum(t, axis=1, keepdims=True) + b_ref[...]
    o_ref[...] = 1.0 / (1.0 + jnp.exp(-t))


def _tail(pre5, scale5, shift5, tail_w, tail_b):
    N = pre5.shape[0]
    wt = tail_w.reshape(3, 3, 1024, 4, 4)
    w_eff = jnp.zeros((6, 6, 1024), jnp.float32)
    for oh in range(3):
        for ow in range(3):
            w_eff = w_eff.at[oh:oh + 4, ow:ow + 4, :].add(
                jnp.transpose(wt[oh, ow], (1, 2, 0)))
    x3 = pre5.reshape(N, 36, 1024)
    out = pl.pallas_call(
        _tail_kernel,
        out_shape=jax.ShapeDtypeStruct((N, 1, 1), jnp.float32),
        compiler_params=pltpu.CompilerParams(vmem_limit_bytes=_VMEM_LIMIT),
    )(x3,
      scale5.reshape(1, 1, 1024),
      shift5.reshape(1, 1, 1024),
      w_eff.reshape(1, 36, 1024),
      tail_b.reshape(1, 1, 1))
    return out.reshape(N, 1)


def kernel(x, conv1_w_mat, bn1_gamma, bn1_beta, conv2_w_mat, bn2_gamma,
           bn2_beta, conv3_w_mat, bn3_gamma, bn3_beta, conv4_w_mat, bn4_gamma,
           bn4_beta, conv5_w_mat, bn5_gamma, bn5_beta, tail_w, tail_b):
    ws = [conv1_w_mat, conv2_w_mat, conv3_w_mat, conv4_w_mat, conv5_w_mat]
    gammas = [bn1_gamma, bn2_gamma, bn3_gamma, bn4_gamma, bn5_gamma]
    betas = [bn1_beta, bn2_beta, bn3_beta, bn4_beta, bn5_beta]

    N = x.shape[0]
    cur = jnp.transpose(x, (0, 2, 3, 1)).astype(jnp.bfloat16)
    scale_k = shift_k = None
    scale = shift = None
    oh = None
    for li in range(5):
        p = _extract_patches(cur, 4, 4, 2)
        _, oh, ow, K = p.shape
        M = N * oh * ow
        pre, s_part, sq_part = _fused_layer(p.reshape(M, K), ws[li],
                                            scale_k, shift_k)
        C = ws[li].shape[1]
        scale, shift = _bn_coeffs(s_part, sq_part, gammas[li], betas[li], M)
        if li < 4:
            scale_k = _fold_k(scale, C)
            shift_k = _fold_k(shift, C)
            cur = pre.reshape(N, oh, ow, C)
        else:
            pre5 = pre.reshape(N, oh, ow, C)
    return _tail(pre5, scale, shift, tail_w, tail_b)


# speedup vs baseline: 6.5881x; 6.5881x over previous
import functools

import jax
import jax.numpy as jnp
from jax.experimental import pallas as pl
from jax.experimental.pallas import tpu as pltpu


_VMEM_LIMIT = 48 * 1024 * 1024
_EPS = 1e-5


def _l1_kernel(a_ref, w_ref, o_ref, s_ref, sq_ref):
    A = a_ref[...]
    taps = [A[:, qh:qh + 126, qw:qw + 126, :]
            for qh in range(2) for qw in range(2)]
    X = jnp.concatenate(taps, axis=-1).reshape(126 * 126, 48)
    out = jnp.dot(X, w_ref[...], preferred_element_type=jnp.float32)
    s_ref[...] = jnp.sum(out, axis=0, keepdims=True)[None]
    sq_ref[...] = jnp.sum(out * out, axis=0, keepdims=True)[None]
    o_ref[...] = out.reshape(1, 126, 126, 64).astype(o_ref.dtype)


def _layer1(x, w_mat):
    N = x.shape[0]
    xs2d = (x.reshape(N, 3, 127, 2, 127, 2)
            .transpose(0, 2, 4, 3, 5, 1)
            .reshape(N, 127, 127, 12)
            .astype(jnp.bfloat16))
    w1 = (w_mat.reshape(3, 2, 2, 2, 2, 64)
          .transpose(1, 3, 2, 4, 0, 5)
          .reshape(48, 64))
    pre, s, sq = pl.pallas_call(
        _l1_kernel,
        out_shape=(jax.ShapeDtypeStruct((N, 126, 126, 64), jnp.float32),
                   jax.ShapeDtypeStruct((N, 1, 64), jnp.float32),
                   jax.ShapeDtypeStruct((N, 1, 64), jnp.float32)),
        grid_spec=pltpu.PrefetchScalarGridSpec(
            num_scalar_prefetch=0,
            grid=(N,),
            in_specs=[pl.BlockSpec((1, 127, 127, 12), lambda g: (g, 0, 0, 0)),
                      pl.BlockSpec((48, 64), lambda g: (0, 0))],
            out_specs=(pl.BlockSpec((1, 126, 126, 64), lambda g: (g, 0, 0, 0)),
                       pl.BlockSpec((1, 1, 64), lambda g: (g, 0, 0)),
                       pl.BlockSpec((1, 1, 64), lambda g: (g, 0, 0))),
        ),
        compiler_params=pltpu.CompilerParams(
            dimension_semantics=("parallel",),
            vmem_limit_bytes=_VMEM_LIMIT),
    )(xs2d, w1)
    return pre, s, sq


def _conv_kernel(a_ref, w_ref, scale_ref, shift_ref, o_ref, s_ref, sq_ref, *,
                 OH, OW, Cin):
    A = a_ref[...]
    G, H, W2 = A.shape[0], A.shape[1], A.shape[2]
    y = (A.astype(jnp.float32) * scale_ref[...].reshape(1, 1, 1, 2 * Cin)
         + shift_ref[...].reshape(1, 1, 1, 2 * Cin))
    a = jnp.where(y >= 0.0, y, 0.2 * y).astype(jnp.bfloat16)
    a5 = a.reshape(G, H // 2, 2, W2, 2 * Cin)
    out = jnp.zeros((G * OH * OW, w_ref.shape[2]), jnp.float32)
    for r in range(4):
        q, p = divmod(r, 2)
        ar = a5[:, q:q + OH, p]
        for cq in range(2):
            X = ar[:, :, cq:cq + OW, :].reshape(G * OH * OW, 2 * Cin)
            out = out + jnp.dot(X, w_ref[r * 2 + cq],
                                preferred_element_type=jnp.float32)
    s_ref[...] = jnp.sum(out, axis=0, keepdims=True)[None]
    sq_ref[...] = jnp.sum(out * out, axis=0, keepdims=True)[None]
    o_ref[...] = out.reshape(G, OH, OW, -1).astype(o_ref.dtype)


def _conv_layer(pre_in, w_mat, scale, shift, G, ns):
    N, H, W, Cin = pre_in.shape
    Cout = w_mat.shape[1]
    OH, OW = H // 2 - 1, W // 2 - 1
    ng = N // G
    Cb = Cout // ns
    W2 = W // 2
    a_packed = pre_in.reshape(N, H, W2, 2 * Cin)
    scale2 = jnp.tile(scale, 2).reshape(1, 2 * Cin)
    shift2 = jnp.tile(shift, 2).reshape(1, 2 * Cin)
    w2 = (w_mat.reshape(Cin, 4, 2, 2, Cout)
          .transpose(1, 2, 3, 0, 4)
          .reshape(8, 2 * Cin, Cout))
    kern = functools.partial(_conv_kernel, OH=OH, OW=OW, Cin=Cin)
    pre, s, sq = pl.pallas_call(
        kern,
        out_shape=(jax.ShapeDtypeStruct((N, OH, OW, Cout), jnp.float32),
                   jax.ShapeDtypeStruct((ng, 1, Cout), jnp.float32),
                   jax.ShapeDtypeStruct((ng, 1, Cout), jnp.float32)),
        grid_spec=pltpu.PrefetchScalarGridSpec(
            num_scalar_prefetch=0,
            grid=(ng, ns),
            in_specs=[pl.BlockSpec((G, H, W2, 2 * Cin),
                                   lambda g, j: (g, 0, 0, 0)),
                      pl.BlockSpec((8, 2 * Cin, Cb), lambda g, j: (0, 0, j)),
                      pl.BlockSpec((1, 2 * Cin), lambda g, j: (0, 0)),
                      pl.BlockSpec((1, 2 * Cin), lambda g, j: (0, 0))],
            out_specs=(pl.BlockSpec((G, OH, OW, Cb),
                                    lambda g, j: (g, 0, 0, j)),
                       pl.BlockSpec((1, 1, Cb), lambda g, j: (g, 0, j)),
                       pl.BlockSpec((1, 1, Cb), lambda g, j: (g, 0, j))),
        ),
        compiler_params=pltpu.CompilerParams(
            dimension_semantics=("parallel", "parallel"),
            vmem_limit_bytes=_VMEM_LIMIT),
    )(a_packed, w2, scale2, shift2)
    return pre, s, sq


def _bn_coeffs(s_part, sq_part, gamma, beta, M):
    s = jnp.sum(s_part, axis=(0, 1))
    sq = jnp.sum(sq_part, axis=(0, 1))
    inv_m = 1.0 / M
    mean = s * inv_m
    var = jnp.maximum(sq * inv_m - mean * mean, 0.0)
    scale = gamma * jax.lax.rsqrt(var + _EPS)
    shift = beta - mean * scale
    return scale, shift


def _tail_kernel(x_ref, sc_ref, sh_ref, w_ref, b_ref, o_ref):
    y = x_ref[...].astype(jnp.float32) * sc_ref[...] + sh_ref[...]
    a = jnp.where(y >= 0.0, y, 0.2 * y)
    prod = a * w_ref[...]
    t = jnp.sum(prod, axis=2, keepdims=True)
    t = jnp.sum(t, axis=1, keepdims=True) + b_ref[...]
    o_ref[...] = 1.0 / (1.0 + jnp.exp(-t))


def _tail(pre5, scale5, shift5, tail_w, tail_b):
    N = pre5.shape[0]
    wt = tail_w.reshape(3, 3, 1024, 4, 4)
    w_eff = jnp.zeros((6, 6, 1024), jnp.float32)
    for oh in range(3):
        for ow in range(3):
            w_eff = w_eff.at[oh:oh + 4, ow:ow + 4, :].add(
                jnp.transpose(wt[oh, ow], (1, 2, 0)))
    out = pl.pallas_call(
        _tail_kernel,
        out_shape=jax.ShapeDtypeStruct((N, 1, 1), jnp.float32),
        compiler_params=pltpu.CompilerParams(vmem_limit_bytes=_VMEM_LIMIT),
    )(pre5.reshape(N, 36, 1024),
      scale5.reshape(1, 1, 1024),
      shift5.reshape(1, 1, 1024),
      w_eff.reshape(1, 36, 1024),
      tail_b.reshape(1, 1, 1))
    return out.reshape(N, 1)


def kernel(x, conv1_w_mat, bn1_gamma, bn1_beta, conv2_w_mat, bn2_gamma,
           bn2_beta, conv3_w_mat, bn3_gamma, bn3_beta, conv4_w_mat, bn4_gamma,
           bn4_beta, conv5_w_mat, bn5_gamma, bn5_beta, tail_w, tail_b):
    N = x.shape[0]
    pre, s, sq = _layer1(x, conv1_w_mat)
    scale, shift = _bn_coeffs(s, sq, bn1_gamma, bn1_beta, N * 126 * 126)

    layer_cfg = [(conv2_w_mat, bn2_gamma, bn2_beta, 1, 1),
                 (conv3_w_mat, bn3_gamma, bn3_beta, 2, 1),
                 (conv4_w_mat, bn4_gamma, bn4_beta, 4, 1),
                 (conv5_w_mat, bn5_gamma, bn5_beta, 8, 2)]
    for w_mat, gamma, beta, G, ns in layer_cfg:
        pre, s, sq = _conv_layer(pre, w_mat, scale, shift, G, ns)
        M = pre.shape[0] * pre.shape[1] * pre.shape[2]
        scale, shift = _bn_coeffs(s, sq, gamma, beta, M)

    return _tail(pre, scale, shift, tail_w, tail_b)


# speedup vs baseline: 7.0896x; 1.0761x over previous
import functools

import jax
import jax.numpy as jnp
from jax.experimental import pallas as pl
from jax.experimental.pallas import tpu as pltpu


_VMEM_LIMIT = 48 * 1024 * 1024
_EPS = 1e-5


def _l1_kernel(a_ref, w_ref, o_ref, s_ref, sq_ref):
    A = a_ref[...]
    taps = [A[:, qh:qh + 126, qw:qw + 126, :]
            for qh in range(2) for qw in range(2)]
    X = jnp.concatenate(taps, axis=-1).reshape(126 * 126, 48)
    out = jnp.dot(X, w_ref[...], preferred_element_type=jnp.float32)
    s_ref[...] = jnp.sum(out, axis=0, keepdims=True)[None]
    sq_ref[...] = jnp.sum(out * out, axis=0, keepdims=True)[None]
    o_ref[...] = out.reshape(1, 126, 126, 64).astype(o_ref.dtype)


def _layer1(x, w_mat):
    N = x.shape[0]
    xs2d = (x.reshape(N, 3, 127, 2, 127, 2)
            .transpose(0, 2, 4, 3, 5, 1)
            .reshape(N, 127, 127, 12)
            .astype(jnp.bfloat16))
    w1 = (w_mat.reshape(3, 2, 2, 2, 2, 64)
          .transpose(1, 3, 2, 4, 0, 5)
          .reshape(48, 64))
    pre, s, sq = pl.pallas_call(
        _l1_kernel,
        out_shape=(jax.ShapeDtypeStruct((N, 126, 126, 64), jnp.float32),
                   jax.ShapeDtypeStruct((N, 1, 64), jnp.float32),
                   jax.ShapeDtypeStruct((N, 1, 64), jnp.float32)),
        grid_spec=pltpu.PrefetchScalarGridSpec(
            num_scalar_prefetch=0,
            grid=(N,),
            in_specs=[pl.BlockSpec((1, 127, 127, 12), lambda g: (g, 0, 0, 0)),
                      pl.BlockSpec((48, 64), lambda g: (0, 0))],
            out_specs=(pl.BlockSpec((1, 126, 126, 64), lambda g: (g, 0, 0, 0)),
                       pl.BlockSpec((1, 1, 64), lambda g: (g, 0, 0)),
                       pl.BlockSpec((1, 1, 64), lambda g: (g, 0, 0))),
        ),
        compiler_params=pltpu.CompilerParams(
            dimension_semantics=("parallel",),
            vmem_limit_bytes=_VMEM_LIMIT),
    )(xs2d, w1)
    return pre, s, sq


def _conv_kernel(a_ref, w_ref, scale_ref, shift_ref, o_ref, s_ref, sq_ref, *,
                 OH, OW, Cin):
    A = a_ref[...]
    G, H, W2 = A.shape[0], A.shape[1], A.shape[2]
    y = (A.astype(jnp.float32) * scale_ref[...].reshape(1, 1, 1, 2 * Cin)
         + shift_ref[...].reshape(1, 1, 1, 2 * Cin))
    a = jnp.where(y >= 0.0, y, 0.2 * y).astype(jnp.bfloat16)
    a5 = a.reshape(G, H // 2, 2, W2, 2 * Cin)
    taps = []
    for r in range(4):
        q, p = divmod(r, 2)
        ar = a5[:, q:q + OH, p]
        for cq in range(2):
            taps.append(ar[:, :, cq:cq + OW, :])
    X = jnp.concatenate(taps, axis=-1).reshape(G * OH * OW, 16 * Cin)
    out = jnp.dot(X, w_ref[...], preferred_element_type=jnp.float32)
    s_ref[...] = jnp.sum(out, axis=0, keepdims=True)[None]
    sq_ref[...] = jnp.sum(out * out, axis=0, keepdims=True)[None]
    o_ref[...] = out.reshape(G, OH, OW, -1).astype(o_ref.dtype)


def _conv_layer(pre_in, w_mat, scale, shift, G, ns):
    N, H, W, Cin = pre_in.shape
    Cout = w_mat.shape[1]
    OH, OW = H // 2 - 1, W // 2 - 1
    ng = N // G
    Cb = Cout // ns
    W2 = W // 2
    a_packed = pre_in.reshape(N, H, W2, 2 * Cin)
    scale2 = jnp.tile(scale, 2).reshape(1, 2 * Cin)
    shift2 = jnp.tile(shift, 2).reshape(1, 2 * Cin)
    w2 = (w_mat.reshape(Cin, 4, 2, 2, Cout)
          .transpose(1, 2, 3, 0, 4)
          .reshape(16 * Cin, Cout))
    kern = functools.partial(_conv_kernel, OH=OH, OW=OW, Cin=Cin)
    pre, s, sq = pl.pallas_call(
        kern,
        out_shape=(jax.ShapeDtypeStruct((N, OH, OW, Cout), jnp.float32),
                   jax.ShapeDtypeStruct((ng, 1, Cout), jnp.float32),
                   jax.ShapeDtypeStruct((ng, 1, Cout), jnp.float32)),
        grid_spec=pltpu.PrefetchScalarGridSpec(
            num_scalar_prefetch=0,
            grid=(ng, ns),
            in_specs=[pl.BlockSpec((G, H, W2, 2 * Cin),
                                   lambda g, j: (g, 0, 0, 0)),
                      pl.BlockSpec((16 * Cin, Cb), lambda g, j: (0, j)),
                      pl.BlockSpec((1, 2 * Cin), lambda g, j: (0, 0)),
                      pl.BlockSpec((1, 2 * Cin), lambda g, j: (0, 0))],
            out_specs=(pl.BlockSpec((G, OH, OW, Cb),
                                    lambda g, j: (g, 0, 0, j)),
                       pl.BlockSpec((1, 1, Cb), lambda g, j: (g, 0, j)),
                       pl.BlockSpec((1, 1, Cb), lambda g, j: (g, 0, j))),
        ),
        compiler_params=pltpu.CompilerParams(
            dimension_semantics=("parallel", "parallel"),
            vmem_limit_bytes=_VMEM_LIMIT),
    )(a_packed, w2, scale2, shift2)
    return pre, s, sq


def _bn_coeffs(s_part, sq_part, gamma, beta, M):
    s = jnp.sum(s_part, axis=(0, 1))
    sq = jnp.sum(sq_part, axis=(0, 1))
    inv_m = 1.0 / M
    mean = s * inv_m
    var = jnp.maximum(sq * inv_m - mean * mean, 0.0)
    scale = gamma * jax.lax.rsqrt(var + _EPS)
    shift = beta - mean * scale
    return scale, shift


def _tail_kernel(x_ref, sc_ref, sh_ref, w_ref, b_ref, o_ref):
    y = x_ref[...].astype(jnp.float32) * sc_ref[...] + sh_ref[...]
    a = jnp.where(y >= 0.0, y, 0.2 * y)
    prod = a * w_ref[...]
    t = jnp.sum(prod, axis=2, keepdims=True)
    t = jnp.sum(t, axis=1, keepdims=True) + b_ref[...]
    o_ref[...] = 1.0 / (1.0 + jnp.exp(-t))


def _tail(pre5, scale5, shift5, tail_w, tail_b):
    N = pre5.shape[0]
    wt = tail_w.reshape(3, 3, 1024, 4, 4)
    w_eff = jnp.zeros((6, 6, 1024), jnp.float32)
    for oh in range(3):
        for ow in range(3):
            w_eff = w_eff.at[oh:oh + 4, ow:ow + 4, :].add(
                jnp.transpose(wt[oh, ow], (1, 2, 0)))
    out = pl.pallas_call(
        _tail_kernel,
        out_shape=jax.ShapeDtypeStruct((N, 1, 1), jnp.float32),
        compiler_params=pltpu.CompilerParams(vmem_limit_bytes=_VMEM_LIMIT),
    )(pre5.reshape(N, 36, 1024),
      scale5.reshape(1, 1, 1024),
      shift5.reshape(1, 1, 1024),
      w_eff.reshape(1, 36, 1024),
      tail_b.reshape(1, 1, 1))
    return out.reshape(N, 1)


def kernel(x, conv1_w_mat, bn1_gamma, bn1_beta, conv2_w_mat, bn2_gamma,
           bn2_beta, conv3_w_mat, bn3_gamma, bn3_beta, conv4_w_mat, bn4_gamma,
           bn4_beta, conv5_w_mat, bn5_gamma, bn5_beta, tail_w, tail_b):
    N = x.shape[0]
    pre, s, sq = _layer1(x, conv1_w_mat)
    scale, shift = _bn_coeffs(s, sq, bn1_gamma, bn1_beta, N * 126 * 126)

    layer_cfg = [(conv2_w_mat, bn2_gamma, bn2_beta, 1, 1),
                 (conv3_w_mat, bn3_gamma, bn3_beta, 2, 1),
                 (conv4_w_mat, bn4_gamma, bn4_beta, 4, 2),
                 (conv5_w_mat, bn5_gamma, bn5_beta, 8, 4)]
    for w_mat, gamma, beta, G, ns in layer_cfg:
        pre, s, sq = _conv_layer(pre, w_mat, scale, shift, G, ns)
        M = pre.shape[0] * pre.shape[1] * pre.shape[2]
        scale, shift = _bn_coeffs(s, sq, gamma, beta, M)

    return _tail(pre, scale, shift, tail_w, tail_b)


# speedup vs baseline: 7.2182x; 1.0181x over previous
import functools

import jax
import jax.numpy as jnp
from jax.experimental import pallas as pl
from jax.experimental.pallas import tpu as pltpu


_VMEM_LIMIT = 48 * 1024 * 1024
_EPS = 1e-5


def _l1_kernel(a_ref, w_ref, o_ref, s_ref, sq_ref):
    A = a_ref[...]
    taps = [A[:, qh:qh + 126, qw:qw + 126, :]
            for qh in range(2) for qw in range(2)]
    X = jnp.concatenate(taps, axis=-1).reshape(126 * 126, 48)
    out = jnp.dot(X, w_ref[...], preferred_element_type=jnp.float32)
    s_ref[...] = jnp.sum(out, axis=0, keepdims=True)[None]
    sq_ref[...] = jnp.sum(out * out, axis=0, keepdims=True)[None]
    o_ref[...] = out.reshape(1, 126, 126, 64).astype(o_ref.dtype)


def _layer1(x, w_mat):
    N = x.shape[0]
    xs2d = (x.reshape(N, 3, 127, 2, 127, 2)
            .transpose(0, 2, 4, 3, 5, 1)
            .reshape(N, 127, 127, 12)
            .astype(jnp.bfloat16))
    w1 = (w_mat.reshape(3, 2, 2, 2, 2, 64)
          .transpose(1, 3, 2, 4, 0, 5)
          .reshape(48, 64))
    pre, s, sq = pl.pallas_call(
        _l1_kernel,
        out_shape=(jax.ShapeDtypeStruct((N, 126, 126, 64), jnp.bfloat16),
                   jax.ShapeDtypeStruct((N, 1, 64), jnp.float32),
                   jax.ShapeDtypeStruct((N, 1, 64), jnp.float32)),
        grid_spec=pltpu.PrefetchScalarGridSpec(
            num_scalar_prefetch=0,
            grid=(N,),
            in_specs=[pl.BlockSpec((1, 127, 127, 12), lambda g: (g, 0, 0, 0)),
                      pl.BlockSpec((48, 64), lambda g: (0, 0))],
            out_specs=(pl.BlockSpec((1, 126, 126, 64), lambda g: (g, 0, 0, 0)),
                       pl.BlockSpec((1, 1, 64), lambda g: (g, 0, 0)),
                       pl.BlockSpec((1, 1, 64), lambda g: (g, 0, 0))),
        ),
        compiler_params=pltpu.CompilerParams(
            dimension_semantics=("parallel",),
            vmem_limit_bytes=_VMEM_LIMIT),
    )(xs2d, w1)
    return pre, s, sq


def _conv_kernel(a_ref, w_ref, scale_ref, shift_ref, o_ref, s_ref, sq_ref, *,
                 OH, OW, Cin):
    A = a_ref[...]
    G, H, W2 = A.shape[0], A.shape[1], A.shape[2]
    y = (A.astype(jnp.float32) * scale_ref[...].reshape(1, 1, 1, 2 * Cin)
         + shift_ref[...].reshape(1, 1, 1, 2 * Cin))
    a = jnp.where(y >= 0.0, y, 0.2 * y).astype(jnp.bfloat16)
    a5 = a.reshape(G, H // 2, 2, W2, 2 * Cin)
    taps = []
    for r in range(4):
        q, p = divmod(r, 2)
        ar = a5[:, q:q + OH, p]
        for cq in range(2):
            taps.append(ar[:, :, cq:cq + OW, :])
    X = jnp.concatenate(taps, axis=-1).reshape(G * OH * OW, 16 * Cin)
    out = jnp.dot(X, w_ref[...], preferred_element_type=jnp.float32)
    s_ref[...] = jnp.sum(out, axis=0, keepdims=True)[None]
    sq_ref[...] = jnp.sum(out * out, axis=0, keepdims=True)[None]
    o_ref[...] = out.reshape(G, OH, OW, -1).astype(o_ref.dtype)


def _conv_layer(pre_in, w_mat, scale, shift, G, ns):
    N, H, W, Cin = pre_in.shape
    Cout = w_mat.shape[1]
    OH, OW = H // 2 - 1, W // 2 - 1
    ng = N // G
    Cb = Cout // ns
    W2 = W // 2
    a_packed = pre_in.reshape(N, H, W2, 2 * Cin)
    scale2 = jnp.tile(scale, 2).reshape(1, 2 * Cin)
    shift2 = jnp.tile(shift, 2).reshape(1, 2 * Cin)
    w2 = (w_mat.reshape(Cin, 4, 2, 2, Cout)
          .transpose(1, 2, 3, 0, 4)
          .reshape(16 * Cin, Cout))
    kern = functools.partial(_conv_kernel, OH=OH, OW=OW, Cin=Cin)
    pre, s, sq = pl.pallas_call(
        kern,
        out_shape=(jax.ShapeDtypeStruct((N, OH, OW, Cout), jnp.float32),
                   jax.ShapeDtypeStruct((ng, 1, Cout), jnp.float32),
                   jax.ShapeDtypeStruct((ng, 1, Cout), jnp.float32)),
        grid_spec=pltpu.PrefetchScalarGridSpec(
            num_scalar_prefetch=0,
            grid=(ng, ns),
            in_specs=[pl.BlockSpec((G, H, W2, 2 * Cin),
                                   lambda g, j: (g, 0, 0, 0)),
                      pl.BlockSpec((16 * Cin, Cb), lambda g, j: (0, j)),
                      pl.BlockSpec((1, 2 * Cin), lambda g, j: (0, 0)),
                      pl.BlockSpec((1, 2 * Cin), lambda g, j: (0, 0))],
            out_specs=(pl.BlockSpec((G, OH, OW, Cb),
                                    lambda g, j: (g, 0, 0, j)),
                       pl.BlockSpec((1, 1, Cb), lambda g, j: (g, 0, j)),
                       pl.BlockSpec((1, 1, Cb), lambda g, j: (g, 0, j))),
        ),
        compiler_params=pltpu.CompilerParams(
            dimension_semantics=("parallel", "parallel"),
            vmem_limit_bytes=_VMEM_LIMIT),
    )(a_packed, w2, scale2, shift2)
    return pre, s, sq


def _bn_coeffs(s_part, sq_part, gamma, beta, M):
    s = jnp.sum(s_part, axis=(0, 1))
    sq = jnp.sum(sq_part, axis=(0, 1))
    inv_m = 1.0 / M
    mean = s * inv_m
    var = jnp.maximum(sq * inv_m - mean * mean, 0.0)
    scale = gamma * jax.lax.rsqrt(var + _EPS)
    shift = beta - mean * scale
    return scale, shift


def _tail_kernel(x_ref, sc_ref, sh_ref, w_ref, b_ref, o_ref):
    y = x_ref[...].astype(jnp.float32) * sc_ref[...] + sh_ref[...]
    a = jnp.where(y >= 0.0, y, 0.2 * y)
    prod = a * w_ref[...]
    t = jnp.sum(prod, axis=2, keepdims=True)
    t = jnp.sum(t, axis=1, keepdims=True) + b_ref[...]
    o_ref[...] = 1.0 / (1.0 + jnp.exp(-t))


def _tail(pre5, scale5, shift5, tail_w, tail_b):
    N = pre5.shape[0]
    wt = tail_w.reshape(3, 3, 1024, 4, 4)
    w_eff = jnp.zeros((6, 6, 1024), jnp.float32)
    for oh in range(3):
        for ow in range(3):
            w_eff = w_eff.at[oh:oh + 4, ow:ow + 4, :].add(
                jnp.transpose(wt[oh, ow], (1, 2, 0)))
    out = pl.pallas_call(
        _tail_kernel,
        out_shape=jax.ShapeDtypeStruct((N, 1, 1), jnp.float32),
        compiler_params=pltpu.CompilerParams(vmem_limit_bytes=_VMEM_LIMIT),
    )(pre5.reshape(N, 36, 1024),
      scale5.reshape(1, 1, 1024),
      shift5.reshape(1, 1, 1024),
      w_eff.reshape(1, 36, 1024),
      tail_b.reshape(1, 1, 1))
    return out.reshape(N, 1)


def kernel(x, conv1_w_mat, bn1_gamma, bn1_beta, conv2_w_mat, bn2_gamma,
           bn2_beta, conv3_w_mat, bn3_gamma, bn3_beta, conv4_w_mat, bn4_gamma,
           bn4_beta, conv5_w_mat, bn5_gamma, bn5_beta, tail_w, tail_b):
    N = x.shape[0]
    pre, s, sq = _layer1(x, conv1_w_mat)
    scale, shift = _bn_coeffs(s, sq, bn1_gamma, bn1_beta, N * 126 * 126)

    layer_cfg = [(conv2_w_mat, bn2_gamma, bn2_beta, 1, 1),
                 (conv3_w_mat, bn3_gamma, bn3_beta, 2, 1),
                 (conv4_w_mat, bn4_gamma, bn4_beta, 4, 2),
                 (conv5_w_mat, bn5_gamma, bn5_beta, 8, 4)]
    for w_mat, gamma, beta, G, ns in layer_cfg:
        pre, s, sq = _conv_layer(pre, w_mat, scale, shift, G, ns)
        M = pre.shape[0] * pre.shape[1] * pre.shape[2]
        scale, shift = _bn_coeffs(s, sq, gamma, beta, M)

    return _tail(pre, scale, shift, tail_w, tail_b)
